# Initial kernel scaffold; baseline (speedup 1.0000x reference)
import jax
import jax.numpy as jnp
from jax.experimental import pallas as pl


def kernel(input_ids, W):
    raise NotImplementedError("write your pallas kernel here")


# speedup vs baseline: 4.1369x; 4.1369x over previous
import functools

import jax
import jax.numpy as jnp
from jax import lax
from jax.experimental import pallas as pl
from jax.experimental.pallas import tpu as pltpu
from jax.experimental.pallas import tpu_sc as plsc

_COMPLEMENT = (0, 1, 2, 3, 4, 5, 6, 10, 9, 8, 7, 11)


def _table_body(w_ref, out_ref):
    w = w_ref[...]
    d = w.shape[1]
    out_ref[:, :d] = w
    rc = jnp.concatenate([w_ref[c:c + 1, :] for c in _COMPLEMENT], axis=0)
    ri = lax.broadcasted_iota(jnp.int32, (d, d), 0)
    ci = lax.broadcasted_iota(jnp.int32, (d, d), 1)
    rev = jnp.where(ri + ci == d - 1, 1.0, 0.0).astype(w.dtype)
    out_ref[:, d:] = jnp.dot(rc, rev, preferred_element_type=jnp.float32)


def _build_table(W):
    v, d = W.shape
    return pl.pallas_call(
        _table_body,
        out_shape=jax.ShapeDtypeStruct((v, 2 * d), W.dtype),
    )(W)


@functools.lru_cache(maxsize=None)
def _make_gather(n, d2):
    info = plsc.get_sparse_core_info()
    nc, ns = info.num_cores, info.num_subcores
    nw = nc * ns
    per_w = n // nw
    assert per_w * nw == n
    chunk = 128
    nch = per_w // chunk
    assert nch * chunk == per_w
    mesh = plsc.VectorSubcoreMesh(core_axis_name="c", subcore_axis_name="s")

    @functools.partial(
        pl.kernel,
        mesh=mesh,
        out_type=jax.ShapeDtypeStruct((n, d2), jnp.float32),
        scratch_types=[
            pltpu.VMEM((per_w,), jnp.int32),
            pltpu.VMEM((chunk, d2), jnp.float32),
            pltpu.SemaphoreType.DMA,
        ],
    )
    def gk(table_hbm, idx_hbm, out_hbm, idx_v, rows_v, sem):
        wid = lax.axis_index("s") * nc + lax.axis_index("c")
        base = wid * per_w
        pltpu.sync_copy(idx_hbm.at[pl.ds(base, per_w)], idx_v)
        for j in range(nch):
            pltpu.async_copy(
                table_hbm.at[idx_v.at[pl.ds(j * chunk, chunk)]], rows_v, sem
            ).wait()
            pltpu.sync_copy(rows_v, out_hbm.at[pl.ds(base + j * chunk, chunk)])

    return gk


def kernel(input_ids, W):
    b, l = input_ids.shape
    v, d = W.shape
    table = _build_table(W)
    ids = input_ids.reshape(b * l)
    out = _make_gather(b * l, 2 * d)(table, ids)
    return out.reshape(b, l, 2 * d)


# speedup vs baseline: 4.2027x; 1.0159x over previous
import functools

import jax
import jax.numpy as jnp
from jax import lax
from jax.experimental import pallas as pl
from jax.experimental.pallas import tpu as pltpu
from jax.experimental.pallas import tpu_sc as plsc

_COMPLEMENT = (0, 1, 2, 3, 4, 5, 6, 10, 9, 8, 7, 11)


def _table_body(w_ref, out_ref):
    w = w_ref[...]
    d = w.shape[1]
    out_ref[:, :d] = w
    rc = jnp.concatenate([w_ref[c:c + 1, :] for c in _COMPLEMENT], axis=0)
    ri = lax.broadcasted_iota(jnp.int32, (d, d), 0)
    ci = lax.broadcasted_iota(jnp.int32, (d, d), 1)
    rev = jnp.where(ri + ci == d - 1, 1.0, 0.0).astype(w.dtype)
    out_ref[:, d:] = jnp.dot(rc, rev, preferred_element_type=jnp.float32)


def _build_table(W):
    v, d = W.shape
    return pl.pallas_call(
        _table_body,
        out_shape=jax.ShapeDtypeStruct((v, 2 * d), W.dtype),
    )(W)


@functools.lru_cache(maxsize=None)
def _make_gather(n, d2):
    info = plsc.get_sparse_core_info()
    nc, ns = info.num_cores, info.num_subcores
    nw = nc * ns
    per_w = n // nw
    assert per_w * nw == n
    chunk = 64
    nch = per_w // chunk
    assert nch * chunk == per_w
    npairs = nch // 2
    assert npairs * 2 == nch
    mesh = plsc.VectorSubcoreMesh(core_axis_name="c", subcore_axis_name="s")

    @functools.partial(
        pl.kernel,
        mesh=mesh,
        out_type=jax.ShapeDtypeStruct((n, d2), jnp.float32),
        scratch_types=[
            pltpu.VMEM((per_w,), jnp.int32),
            pltpu.VMEM((chunk, d2), jnp.float32),
            pltpu.VMEM((chunk, d2), jnp.float32),
            pltpu.SemaphoreType.DMA,
            pltpu.SemaphoreType.DMA,
            pltpu.SemaphoreType.DMA,
            pltpu.SemaphoreType.DMA,
        ],
    )
    def gk(table_hbm, idx_hbm, out_hbm, idx_v, buf0, buf1, sg0, sg1, sw0, sw1):
        wid = lax.axis_index("s") * nc + lax.axis_index("c")
        base = wid * per_w
        pltpu.sync_copy(idx_hbm.at[pl.ds(base, per_w)], idx_v)

        def g_start(j, buf, sem):
            pltpu.async_copy(
                table_hbm.at[idx_v.at[pl.ds(j * chunk, chunk)]], buf, sem
            )

        def g_wait(buf, sem):
            pltpu.make_async_copy(
                table_hbm.at[idx_v.at[pl.ds(0, chunk)]], buf, sem
            ).wait()

        def w_start(j, buf, sem):
            pltpu.async_copy(buf, out_hbm.at[pl.ds(base + j * chunk, chunk)], sem)

        def w_wait(buf, sem):
            pltpu.make_async_copy(buf, out_hbm.at[pl.ds(base, chunk)], sem).wait()

        g_start(0, buf0, sg0)
        g_start(1, buf1, sg1)

        def body(i, carry):
            j0 = 2 * i
            g_wait(buf0, sg0)
            w_start(j0, buf0, sw0)
            g_wait(buf1, sg1)
            w_start(j0 + 1, buf1, sw1)

            @pl.when(i + 1 < npairs)
            def _():
                w_wait(buf0, sw0)
                g_start(j0 + 2, buf0, sg0)
                w_wait(buf1, sw1)
                g_start(j0 + 3, buf1, sg1)

            return carry

        lax.fori_loop(0, npairs, body, 0)
        w_wait(buf0, sw0)
        w_wait(buf1, sw1)

    return gk


def kernel(input_ids, W):
    b, l = input_ids.shape
    v, d = W.shape
    table = _build_table(W)
    ids = input_ids.reshape(b * l)
    out = _make_gather(b * l, 2 * d)(table, ids)
    return out.reshape(b, l, 2 * d)
